# baseline (speedup 1.0000x reference)
import functools

import jax
import jax.numpy as jnp
from jax import lax
from jax.experimental import pallas as pl
from jax.experimental.pallas import tpu as pltpu
from jax.experimental.pallas import tpu_sc as plsc

N_GROUP = 4
INPUT_DIM_GROUP = 250000
OUT_DIM = 16
B = 16384
S = 50
NW = 32
BPW = B // NW
IDX_COLS = 128
G_PER_S = BPW // IDX_COLS
V_PER_S = BPW // 16

_mesh = plsc.VectorSubcoreMesh(core_axis_name="c", subcore_axis_name="s")

R = N_GROUP * INPUT_DIM_GROUP
RPW = 31248
TK = 2496
TKP = TK + 1
N_TBLK = RPW // TK
T_TAIL = RPW - N_TBLK * TK
T_REM = R - NW * RPW
T_UNROLL = 8


@functools.partial(
    pl.kernel,
    out_type=jax.ShapeDtypeStruct((R, OUT_DIM), jnp.float32),
    mesh=_mesh,
    scratch_types=[
        pltpu.VMEM((OUT_DIM, TKP), jnp.float32),
        pltpu.VMEM((TK, OUT_DIM), jnp.float32),
    ],
    compiler_params=pltpu.CompilerParams(
        needs_layout_passes=False, use_tc_tiling_on_sc=False
    ),
)
def _sc_transpose(tabt_hbm, out_hbm, col_v, row_v):
    nc = lax.axis_size("c")
    wid = lax.axis_index("s") * nc + lax.axis_index("c")
    r0w = wid * RPW
    lanes = lax.iota(jnp.int32, 16)
    zeros = lanes * 0

    def do_block(r0, nrows):
        pltpu.sync_copy(
            tabt_hbm.at[pl.ds(0, OUT_DIM), pl.ds(r0, nrows)],
            col_v.at[pl.ds(0, OUT_DIM), pl.ds(0, nrows)],
        )

        def rows(j, _):
            for u in range(T_UNROLL):
                r = j * T_UNROLL + u
                row_v[r, pl.ds(0, OUT_DIM)] = plsc.load_gather(
                    col_v, [lanes, zeros + r]
                )
            return 0
        lax.fori_loop(0, nrows // T_UNROLL, rows, 0)

        pltpu.sync_copy(
            row_v.at[pl.ds(0, nrows)], out_hbm.at[pl.ds(r0, nrows)]
        )

    for k in range(N_TBLK):
        do_block(r0w + k * TK, TK)
    do_block(r0w + N_TBLK * TK, T_TAIL)

    @pl.when(wid == 0)
    def _():
        do_block(NW * RPW, T_REM)


@functools.partial(
    pl.kernel,
    out_type=jax.ShapeDtypeStruct((1, S, OUT_DIM, B), jnp.float32),
    mesh=_mesh,
    scratch_types=[
        pltpu.VMEM((1, BPW), jnp.int32),
        pltpu.VMEM((1, S, BPW), jnp.int32),
        pltpu.VMEM((2, G_PER_S, IDX_COLS), jnp.int32),
        pltpu.VMEM((2, BPW, OUT_DIM), jnp.float32),
        pltpu.VMEM((1, 1, OUT_DIM, BPW + 1), jnp.float32),
        pltpu.SemaphoreType.DMA,
        pltpu.SemaphoreType.DMA,
    ],
    compiler_params=pltpu.CompilerParams(
        needs_layout_passes=False, use_tc_tiling_on_sc=False
    ),
)
def _sc_gather(idx_hbm, grp_hbm, table_hbm, out_hbm,
               grp_v, idx_v, gidx_v, rows_v, rowt_v, sem0, sem1):
    nc = lax.axis_size("c")
    wid = lax.axis_index("s") * nc + lax.axis_index("c")
    b0 = wid * BPW

    pltpu.sync_copy(grp_hbm.at[pl.ds(0, 1), pl.ds(b0, BPW)], grp_v)
    pltpu.sync_copy(idx_hbm.at[pl.ds(0, 1), pl.ds(0, S), pl.ds(b0, BPW)], idx_v)

    lanes = lax.iota(jnp.int32, 16)
    zeros = lanes * 0
    sems = (sem0, sem1)

    def remap(s, buf):
        def body(j, _):
            x = idx_v[0, s, pl.ds(j * 16, 16)]
            g = grp_v[0, pl.ds(j * 16, 16)]
            gidx_v[buf, j // 8, pl.ds((j % 8) * 16, 16)] = x + g * INPUT_DIM_GROUP
            return 0
        lax.fori_loop(0, V_PER_S, body, 0)

    def fire(buf):
        for j in range(G_PER_S):
            pltpu.async_copy(
                table_hbm.at[gidx_v.at[buf, j]],
                rows_v.at[buf, pl.ds(j * IDX_COLS, IDX_COLS)],
                sems[buf],
            )

    def drain_and_write(s, buf):
        pltpu.make_async_copy(
            table_hbm.at[pl.ds(0, BPW)], rows_v.at[buf], sems[buf]
        ).wait()

        def tp(j, _):
            for u in range(8):
                b = j * 8 + u
                v = rows_v[buf, b, pl.ds(0, OUT_DIM)]
                plsc.store_scatter(rowt_v, [zeros, zeros, lanes, zeros + b], v)
            return 0
        lax.fori_loop(0, BPW // 8, tp, 0)

        pltpu.sync_copy(
            rowt_v.at[pl.ds(0, 1), pl.ds(0, 1), pl.ds(0, OUT_DIM), pl.ds(0, BPW)],
            out_hbm.at[pl.ds(0, 1), pl.ds(s, 1), pl.ds(0, OUT_DIM), pl.ds(b0, BPW)],
        )

    remap(0, 0)
    fire(0)
    remap(1, 1)
    fire(1)

    def step(k, _):
        s = 2 * k
        drain_and_write(s, 0)
        remap(s + 2, 0)
        fire(0)
        drain_and_write(s + 1, 1)
        remap(s + 3, 1)
        fire(1)
        return 0

    lax.fori_loop(0, S // 2 - 1, step, 0)
    drain_and_write(S - 2, 0)
    drain_and_write(S - 1, 1)


def kernel(indices, group, table):
    idx_t = jnp.transpose(indices, (1, 2, 0))
    grp_t = jnp.transpose(group, (1, 0))
    table_rm = _sc_transpose(jnp.transpose(table, (1, 0)))
    out_t = _sc_gather(idx_t, grp_t, table_rm)
    return jnp.transpose(out_t, (3, 0, 1, 2))


# speedup vs baseline: 2.6374x; 2.6374x over previous
import functools

import jax
import jax.numpy as jnp
from jax import lax
from jax.experimental import pallas as pl
from jax.experimental.pallas import tpu as pltpu
from jax.experimental.pallas import tpu_sc as plsc

N_GROUP = 4
INPUT_DIM_GROUP = 250000
OUT_DIM = 16
B = 16384
S = 50
NW = 32
BPW = B // NW
IDX_COLS = 128
G_PER_S = BPW // IDX_COLS
V_PER_S = BPW // 16

_mesh = plsc.VectorSubcoreMesh(core_axis_name="c", subcore_axis_name="s")


@functools.partial(
    pl.kernel,
    out_type=jax.ShapeDtypeStruct((1, S, OUT_DIM, B), jnp.float32),
    mesh=_mesh,
    scratch_types=[
        pltpu.VMEM((1, BPW), jnp.int32),
        pltpu.VMEM((1, S, BPW), jnp.int32),
        pltpu.VMEM((2, G_PER_S, IDX_COLS), jnp.int32),
        pltpu.VMEM((2, BPW, OUT_DIM), jnp.float32),
        pltpu.VMEM((1, 1, OUT_DIM, BPW + 1), jnp.float32),
        pltpu.SemaphoreType.DMA,
        pltpu.SemaphoreType.DMA,
    ],
    compiler_params=pltpu.CompilerParams(
        needs_layout_passes=False, use_tc_tiling_on_sc=False
    ),
)
def _sc_gather(idx_hbm, grp_hbm, table_hbm, out_hbm,
               grp_v, idx_v, gidx_v, rows_v, rowt_v, sem0, sem1):
    nc = lax.axis_size("c")
    wid = lax.axis_index("s") * nc + lax.axis_index("c")
    b0 = wid * BPW

    pltpu.sync_copy(grp_hbm.at[pl.ds(0, 1), pl.ds(b0, BPW)], grp_v)
    pltpu.sync_copy(idx_hbm.at[pl.ds(0, 1), pl.ds(0, S), pl.ds(b0, BPW)], idx_v)

    lanes = lax.iota(jnp.int32, 16)
    zeros = lanes * 0
    sems = (sem0, sem1)

    def remap(s, buf):
        def body(j, _):
            x = idx_v[0, s, pl.ds(j * 16, 16)]
            g = grp_v[0, pl.ds(j * 16, 16)]
            gidx_v[buf, j // 8, pl.ds((j % 8) * 16, 16)] = x + g * INPUT_DIM_GROUP
            return 0
        lax.fori_loop(0, V_PER_S, body, 0)

    def fire(buf):
        for j in range(G_PER_S):
            pltpu.async_copy(
                table_hbm.at[gidx_v.at[buf, j]],
                rows_v.at[buf, pl.ds(j * IDX_COLS, IDX_COLS)],
                sems[buf],
            )

    def drain_and_write(s, buf):
        pltpu.make_async_copy(
            table_hbm.at[pl.ds(0, BPW)], rows_v.at[buf], sems[buf]
        ).wait()

        def tp(j, _):
            for u in range(8):
                b = j * 8 + u
                v = rows_v[buf, b, pl.ds(0, OUT_DIM)]
                plsc.store_scatter(rowt_v, [zeros, zeros, lanes, zeros + b], v)
            return 0
        lax.fori_loop(0, BPW // 8, tp, 0)

        pltpu.sync_copy(
            rowt_v.at[pl.ds(0, 1), pl.ds(0, 1), pl.ds(0, OUT_DIM), pl.ds(0, BPW)],
            out_hbm.at[pl.ds(0, 1), pl.ds(s, 1), pl.ds(0, OUT_DIM), pl.ds(b0, BPW)],
        )

    remap(0, 0)
    fire(0)
    remap(1, 1)
    fire(1)

    def step(k, _):
        s = 2 * k
        drain_and_write(s, 0)
        remap(s + 2, 0)
        fire(0)
        drain_and_write(s + 1, 1)
        remap(s + 3, 1)
        fire(1)
        return 0

    lax.fori_loop(0, S // 2 - 1, step, 0)
    drain_and_write(S - 2, 0)
    drain_and_write(S - 1, 1)


def kernel(indices, group, table):
    idx_t = jnp.transpose(indices, (1, 2, 0))
    grp_t = jnp.transpose(group, (1, 0))
    out_t = _sc_gather(idx_t, grp_t, table)
    return jnp.transpose(out_t, (3, 0, 1, 2))


# speedup vs baseline: 2.7119x; 1.0283x over previous
import functools

import jax
import jax.numpy as jnp
from jax import lax
from jax.experimental import pallas as pl
from jax.experimental.pallas import tpu as pltpu
from jax.experimental.pallas import tpu_sc as plsc

N_GROUP = 4
INPUT_DIM_GROUP = 250000
OUT_DIM = 16
B = 16384
S = 50
NW = 32
BPW = B // NW
IDX_COLS = 128
G_PER_S = BPW // IDX_COLS
V_PER_S = BPW // 16

_mesh = plsc.VectorSubcoreMesh(core_axis_name="c", subcore_axis_name="s")


@functools.partial(
    pl.kernel,
    out_type=jax.ShapeDtypeStruct((1, S, OUT_DIM, B), jnp.float32),
    mesh=_mesh,
    scratch_types=[
        pltpu.VMEM((1, BPW), jnp.int32),
        pltpu.VMEM((1, S, BPW), jnp.int32),
        pltpu.VMEM((2, G_PER_S, IDX_COLS), jnp.int32),
        pltpu.VMEM((2, BPW, OUT_DIM), jnp.float32),
        pltpu.VMEM((2, 1, 1, OUT_DIM, BPW + 1), jnp.float32),
        pltpu.SemaphoreType.DMA,
        pltpu.SemaphoreType.DMA,
        pltpu.SemaphoreType.DMA,
        pltpu.SemaphoreType.DMA,
    ],
    compiler_params=pltpu.CompilerParams(
        needs_layout_passes=False, use_tc_tiling_on_sc=False
    ),
)
def _sc_gather(idx_hbm, grp_hbm, table_hbm, out_hbm,
               grp_v, idx_v, gidx_v, rows_v, rowt_v,
               sem0, sem1, semw0, semw1):
    nc = lax.axis_size("c")
    wid = lax.axis_index("s") * nc + lax.axis_index("c")
    b0 = wid * BPW

    pltpu.sync_copy(grp_hbm.at[pl.ds(0, 1), pl.ds(b0, BPW)], grp_v)
    pltpu.sync_copy(idx_hbm.at[pl.ds(0, 1), pl.ds(0, S), pl.ds(b0, BPW)], idx_v)

    lanes = lax.iota(jnp.int32, 16)
    zeros = lanes * 0
    sems = (sem0, sem1)
    semws = (semw0, semw1)

    def remap(s, buf):
        def body(j, _):
            x = idx_v[0, s, pl.ds(j * 16, 16)]
            g = grp_v[0, pl.ds(j * 16, 16)]
            gidx_v[buf, j // 8, pl.ds((j % 8) * 16, 16)] = x + g * INPUT_DIM_GROUP
            return 0
        lax.fori_loop(0, V_PER_S, body, 0)

    def fire(buf):
        for j in range(G_PER_S):
            pltpu.async_copy(
                table_hbm.at[gidx_v.at[buf, j]],
                rows_v.at[buf, pl.ds(j * IDX_COLS, IDX_COLS)],
                sems[buf],
            )

    def drain(buf):
        pltpu.make_async_copy(
            table_hbm.at[pl.ds(0, BPW)], rows_v.at[buf], sems[buf]
        ).wait()

    def out_slice(s):
        return out_hbm.at[
            pl.ds(0, 1), pl.ds(s, 1), pl.ds(0, OUT_DIM), pl.ds(b0, BPW)
        ]

    def rowt_slice(buf):
        return rowt_v.at[
            buf, pl.ds(0, 1), pl.ds(0, 1), pl.ds(0, OUT_DIM), pl.ds(0, BPW)
        ]

    def wait_write(buf):
        pltpu.make_async_copy(rowt_slice(buf), out_slice(0), semws[buf]).wait()

    def transpose_write(s, buf):
        def tp(j, _):
            for u in range(8):
                b = j * 8 + u
                v = rows_v[buf, b, pl.ds(0, OUT_DIM)]
                plsc.store_scatter(
                    rowt_v, [zeros + buf, zeros, zeros, lanes, zeros + b], v
                )
            return 0
        lax.fori_loop(0, BPW // 8, tp, 0)
        pltpu.async_copy(rowt_slice(buf), out_slice(s), semws[buf])

    remap(0, 0)
    fire(0)
    remap(1, 1)
    fire(1)

    def step(k, _):
        for h in range(2):
            s = 2 * k + h
            drain(h)
            remap(s + 2, h)
            fire(h)

            @pl.when(k > 0)
            def _():
                wait_write(h)

            transpose_write(s, h)
        return 0

    lax.fori_loop(0, S // 2 - 1, step, 0)
    for h in range(2):
        s = S - 2 + h
        drain(h)
        wait_write(h)
        transpose_write(s, h)
    wait_write(0)
    wait_write(1)


def kernel(indices, group, table):
    idx_t = jnp.transpose(indices, (1, 2, 0))
    grp_t = jnp.transpose(group, (1, 0))
    out_t = _sc_gather(idx_t, grp_t, table)
    return jnp.transpose(out_t, (3, 0, 1, 2))
